# baseline (speedup 1.0000x reference)
import functools

import jax
import jax.numpy as jnp
from jax import lax
from jax.experimental import pallas as pl
from jax.experimental.pallas import tpu as pltpu
from jax.experimental.pallas import tpu_sc as plsc

D_MODEL = 512
S = 64
D2 = D_MODEL // 2

B_SC = 8
WPB = S * S * D_MODEL
NC, NS = 2, 16
NW = NC * NS
PSLICE = WPB // NW
CHUNK = 32768
NCHUNK = PSLICE // CHUNK


def _build_pos_kernel(row_ref, col_ref, pos_ref):
    pos_ref[:, :, :D2] = jnp.broadcast_to(col_ref[...][None, :, :], (S, S, D2))
    pos_ref[:, :, D2:] = jnp.broadcast_to(row_ref[...][:, None, :], (S, S, D2))


def _add_pos_kernel(x_ref, row_ref, col_ref, out_ref):
    pos_col = jnp.broadcast_to(col_ref[...][None, :, :], (S, S, D2))
    pos_row = jnp.broadcast_to(row_ref[...][:, None, :], (S, S, D2))
    out_ref[0, :, :, :D2] = x_ref[0, :, :, :D2] + pos_col
    out_ref[0, :, :, D2:] = x_ref[0, :, :, D2:] + pos_row


ROWS_PER_BATCH = S * S
PROWS = ROWS_PER_BATCH // NW
CROWS = PROWS // 2


def _make_sc_add(b_tc, b_sc):
    mesh = plsc.VectorSubcoreMesh(core_axis_name="c", subcore_axis_name="s")

    @functools.partial(
        pl.kernel,
        mesh=mesh,
        out_type=jax.ShapeDtypeStruct((b_sc * ROWS_PER_BATCH, D_MODEL), jnp.float32),
        scratch_types=[
            pltpu.VMEM((PROWS, D_MODEL), jnp.float32),
            pltpu.VMEM((CROWS, D_MODEL), jnp.float32),
        ],
        compiler_params=pltpu.CompilerParams(use_tc_tiling_on_sc=True),
    )
    def sc_add(x_hbm, pos_hbm, out_hbm, pos_v, buf_v):
        wid = lax.axis_index("s") * NC + lax.axis_index("c")
        base = wid * PROWS
        pltpu.sync_copy(pos_hbm.at[pl.ds(base, PROWS)], pos_v)

        def batch_body(b, carry):
            xrow = (b_tc + b) * ROWS_PER_BATCH + base
            orow = b * ROWS_PER_BATCH + base
            for k in range(2):
                pltpu.sync_copy(
                    x_hbm.at[pl.ds(xrow + k * CROWS, CROWS)], buf_v
                )

                def add_body(t, _):
                    r = t >> 5
                    o = (t & 31) * 16
                    buf_v[r, pl.ds(o, 16)] = buf_v[r, pl.ds(o, 16)] + pos_v[
                        k * CROWS + r, pl.ds(o, 16)
                    ]
                    return 0

                lax.fori_loop(0, CROWS * (D_MODEL // 16), add_body, 0)
                pltpu.sync_copy(
                    buf_v, out_hbm.at[pl.ds(orow + k * CROWS, CROWS)]
                )
            return carry

        lax.fori_loop(0, b_sc, batch_body, 0)

    return sc_add


def kernel(x, row_embed, col_embed):
    B = x.shape[0]
    b_sc = B_SC
    b_tc = B - b_sc
    xt = jnp.transpose(x, (0, 2, 3, 1))

    pos = pl.pallas_call(
        _build_pos_kernel,
        out_shape=jax.ShapeDtypeStruct((S, S, D_MODEL), x.dtype),
    )(row_embed, col_embed)

    out_tc = pl.pallas_call(
        _add_pos_kernel,
        grid=(b_tc,),
        in_specs=[
            pl.BlockSpec((1, S, S, D_MODEL), lambda b: (b, 0, 0, 0)),
            pl.BlockSpec((S, D2), lambda b: (0, 0)),
            pl.BlockSpec((S, D2), lambda b: (0, 0)),
        ],
        out_specs=pl.BlockSpec((1, S, S, D_MODEL), lambda b: (b, 0, 0, 0)),
        out_shape=jax.ShapeDtypeStruct((b_tc, S, S, D_MODEL), x.dtype),
        compiler_params=pltpu.CompilerParams(
            dimension_semantics=("parallel",),
        ),
    )(xt, row_embed, col_embed)

    x2 = xt.reshape(B * S * S, D_MODEL)
    pos2 = pos.reshape(S * S, D_MODEL)
    out_sc = _make_sc_add(b_tc, b_sc)(x2, pos2)

    out_t = jnp.concatenate([out_tc, out_sc.reshape(b_sc, S, S, D_MODEL)], axis=0)
    return jnp.transpose(out_t, (0, 3, 1, 2))


# speedup vs baseline: 2.4961x; 2.4961x over previous
import jax
import jax.numpy as jnp
from jax.experimental import pallas as pl
from jax.experimental.pallas import tpu as pltpu

D_MODEL = 512
S = 64
D2 = D_MODEL // 2
SH = S // 2


def _add_pos_kernel(x_ref, row_ref, col_ref, out_ref):
    h = pl.program_id(1)
    row_half = row_ref[pl.ds(h * SH, SH), :]
    pos_col = jnp.broadcast_to(col_ref[...][None, :, :], (SH, S, D2))
    pos_row = jnp.broadcast_to(row_half[:, None, :], (SH, S, D2))
    out_ref[0, :, :, :D2] = x_ref[0, :, :, :D2] + pos_col
    out_ref[0, :, :, D2:] = x_ref[0, :, :, D2:] + pos_row


def kernel(x, row_embed, col_embed):
    B = x.shape[0]
    xt = jnp.transpose(x, (0, 2, 3, 1))
    out_t = pl.pallas_call(
        _add_pos_kernel,
        grid=(B, 2),
        in_specs=[
            pl.BlockSpec((1, SH, S, D_MODEL), lambda b, h: (b, h, 0, 0)),
            pl.BlockSpec((S, D2), lambda b, h: (0, 0)),
            pl.BlockSpec((S, D2), lambda b, h: (0, 0)),
        ],
        out_specs=pl.BlockSpec((1, SH, S, D_MODEL), lambda b, h: (b, h, 0, 0)),
        out_shape=jax.ShapeDtypeStruct((B, S, S, D_MODEL), x.dtype),
        compiler_params=pltpu.CompilerParams(
            dimension_semantics=("parallel", "parallel"),
        ),
    )(xt, row_embed, col_embed)
    return jnp.transpose(out_t, (0, 3, 1, 2))


# speedup vs baseline: 2.5268x; 1.0123x over previous
import jax
import jax.numpy as jnp
from jax.experimental import pallas as pl
from jax.experimental.pallas import tpu as pltpu

D_MODEL = 512
S = 64
D2 = D_MODEL // 2


def _add_pos_kernel(x_ref, row_ref, col_ref, out_ref):
    pos_col = jnp.broadcast_to(col_ref[...][None, :, :], (S, S, D2))
    pos_row = jnp.broadcast_to(row_ref[...][:, None, :], (S, S, D2))
    out_ref[0, :, :, :D2] = x_ref[0, :, :, :D2] + pos_col
    out_ref[0, :, :, D2:] = x_ref[0, :, :, D2:] + pos_row


def kernel(x, row_embed, col_embed):
    B = x.shape[0]
    xt = jnp.transpose(x, (0, 2, 3, 1))
    out_t = pl.pallas_call(
        _add_pos_kernel,
        grid=(B,),
        in_specs=[
            pl.BlockSpec((1, S, S, D_MODEL), lambda b: (b, 0, 0, 0)),
            pl.BlockSpec((S, D2), lambda b: (0, 0)),
            pl.BlockSpec((S, D2), lambda b: (0, 0)),
        ],
        out_specs=pl.BlockSpec((1, S, S, D_MODEL), lambda b: (b, 0, 0, 0)),
        out_shape=jax.ShapeDtypeStruct((B, S, S, D_MODEL), x.dtype),
        compiler_params=pltpu.CompilerParams(
            dimension_semantics=("parallel",),
        ),
    )(xt, row_embed, col_embed)
    return jnp.transpose(out_t, (0, 3, 1, 2))
